# baseline (speedup 1.0000x reference)
import functools

import jax
import jax.numpy as jnp
from jax import lax
from jax.experimental import pallas as pl
from jax.experimental.pallas import tpu as pltpu
from jax.experimental.pallas import tpu_sc as plsc


def kernel(x_user, x_track_pos, x_track_neg, user_mat, track_mat):
    B = x_user.shape[0]
    V, D = user_mat.shape
    info = plsc.get_sparse_core_info()
    NW = info.num_cores * info.num_subcores
    CG = 8
    NB = NW // (D // CG)
    b = B // NB

    utf = user_mat.T.reshape(-1)
    ttf = track_mat.T.reshape(-1)

    mesh = plsc.VectorSubcoreMesh(core_axis_name="c", subcore_axis_name="s")
    out_sds = jax.ShapeDtypeStruct((D, B), jnp.float32)

    @functools.partial(
        pl.kernel,
        mesh=mesh,
        out_type=(out_sds, out_sds, out_sds),
        scratch_types=(
            [pltpu.VMEM((b,), jnp.int32)]
            + [pltpu.VMEM((b,), jnp.int32) for _ in range(CG)]
            + [pltpu.VMEM((b,), jnp.float32) for _ in range(CG)]
            + [pltpu.SemaphoreType.DMA, pltpu.SemaphoreType.DMA]
        ),
        compiler_params=pltpu.CompilerParams(needs_layout_passes=False),
    )
    def gather3(xu, xp, xn, ut, tt, out_u, out_p, out_n, *scratch):
        idx_v = scratch[0]
        eidx = scratch[1:1 + CG]
        vals = scratch[1 + CG:1 + 2 * CG]
        sem, sem_o = scratch[1 + 2 * CG], scratch[2 + 2 * CG]
        wid = lax.axis_index("s") * info.num_cores + lax.axis_index("c")
        g = wid // NB
        s = wid % NB
        c0 = g * CG
        L = info.num_lanes

        def one(x_hbm, tf, out_hbm):
            pltpu.sync_copy(x_hbm.at[pl.ds(s * b, b)], idx_v)
            copies = []
            for j in range(CG):
                cj = (c0 + j) * V

                def body(k, _, j=j, cj=cj):
                    eidx[j][pl.ds(k * L, L)] = idx_v[pl.ds(k * L, L)] + cj
                    return 0

                lax.fori_loop(0, b // L, body, 0)
                copies.append(pltpu.async_copy(tf.at[eidx[j]], vals[j], sem))
            wb = []
            for j in range(CG):
                copies[j].wait()
                wb.append(pltpu.async_copy(
                    vals[j], out_hbm.at[c0 + j, pl.ds(s * b, b)], sem_o))
            for w in wb:
                w.wait()

        one(xu, ut, out_u)
        one(xp, tt, out_p)
        one(xn, tt, out_n)

    u_t, p_t, n_t = gather3(x_user, x_track_pos, x_track_neg, utf, ttf)
    return (u_t.T, p_t.T, n_t.T)


# speedup vs baseline: 8.9085x; 8.9085x over previous
import functools

import jax
import jax.numpy as jnp
from jax import lax
from jax.experimental import pallas as pl
from jax.experimental.pallas import tpu as pltpu
from jax.experimental.pallas import tpu_sc as plsc


def _gather_call(n_idx_args, B, D, info):
    NW = info.num_cores * info.num_subcores
    b = B // NW
    mesh = plsc.VectorSubcoreMesh(core_axis_name="c", subcore_axis_name="s")
    out_sds = jax.ShapeDtypeStruct((B, D), jnp.float32)

    @functools.partial(
        pl.kernel,
        mesh=mesh,
        out_type=(out_sds,) * n_idx_args,
        scratch_types=(
            [pltpu.VMEM((b,), jnp.int32) for _ in range(n_idx_args)]
            + [pltpu.VMEM((b, D), jnp.float32) for _ in range(n_idx_args)]
            + [pltpu.SemaphoreType.DMA for _ in range(n_idx_args)]
            + [pltpu.SemaphoreType.DMA]
        ),
        compiler_params=pltpu.CompilerParams(use_tc_tiling_on_sc=False),
    )
    def call(*args):
        xs = args[:n_idx_args]
        table = args[n_idx_args]
        outs = args[n_idx_args + 1:2 * n_idx_args + 1]
        rest = args[2 * n_idx_args + 1:]
        idxs = rest[:n_idx_args]
        rows = rest[n_idx_args:2 * n_idx_args]
        sems = rest[2 * n_idx_args:3 * n_idx_args]
        sem_o = rest[3 * n_idx_args]
        wid = lax.axis_index("s") * info.num_cores + lax.axis_index("c")
        sl = pl.ds(wid * b, b)
        gs = []
        for k in range(n_idx_args):
            pltpu.sync_copy(xs[k].at[sl], idxs[k])
            gs.append(pltpu.async_copy(table.at[idxs[k]], rows[k], sems[k]))
        ws = []
        for k in range(n_idx_args):
            gs[k].wait()
            ws.append(pltpu.async_copy(rows[k], outs[k].at[sl], sem_o))
        for w in ws:
            w.wait()

    return call


def kernel(x_user, x_track_pos, x_track_neg, user_mat, track_mat):
    B = x_user.shape[0]
    D = user_mat.shape[1]
    info = plsc.get_sparse_core_info()
    (u,) = _gather_call(1, B, D, info)(x_user, user_mat)
    p, n = _gather_call(2, B, D, info)(x_track_pos, x_track_neg, track_mat)
    return (u, p, n)


# speedup vs baseline: 17.3977x; 1.9529x over previous
import functools

import jax
import jax.numpy as jnp
from jax import lax
from jax.experimental import pallas as pl
from jax.experimental.pallas import tpu as pltpu
from jax.experimental.pallas import tpu_sc as plsc


def kernel(x_user, x_track_pos, x_track_neg, user_mat, track_mat):
    B = x_user.shape[0]
    V, D = user_mat.shape
    info = plsc.get_sparse_core_info()
    NW = info.num_cores * info.num_subcores
    L = info.num_lanes
    b = B // NW
    K = 8

    ut = user_mat.T
    tt = track_mat.T

    mesh = plsc.VectorSubcoreMesh(core_axis_name="c", subcore_axis_name="s")
    out_sds = jax.ShapeDtypeStruct((D, B), jnp.float32)

    @functools.partial(
        pl.kernel,
        mesh=mesh,
        out_type=(out_sds, out_sds, out_sds),
        scratch_types=(
            [pltpu.VMEM((b,), jnp.int32),
             pltpu.VMEM((K * D, 128), jnp.float32),
             pltpu.VMEM((D, b), jnp.float32)]
            + [pltpu.SemaphoreType.DMA for _ in range(K)]
        ),
        compiler_params=pltpu.CompilerParams(needs_layout_passes=False),
    )
    def gather3(xu, xp, xn, ut_r, tt_r, out_u, out_p, out_n,
                idx_v, ring, vals, *sems):
        wid = lax.axis_index("s") * info.num_cores + lax.axis_index("c")
        base = wid * b
        lanes = lax.iota(jnp.int32, L)

        def one(x_hbm, tf, out_hbm):
            pltpu.sync_copy(x_hbm.at[pl.ds(base, b)], idx_v)

            def fire(xi, slot):
                blk = pl.multiple_of((xi >> 7) * 128, 128)
                pltpu.async_copy(
                    tf.at[pl.ds(0, D), pl.ds(blk, 128)],
                    ring.at[pl.ds(slot * D, D), pl.ds(0, 128)],
                    sems[slot])

            def drain_extract(xi, i_dst, slot):
                pltpu.make_async_copy(
                    tf.at[pl.ds(0, D), pl.ds(0, 128)],
                    ring.at[pl.ds(slot * D, D), pl.ds(0, 128)],
                    sems[slot]).wait()
                col = jnp.broadcast_to(xi & 127, (L,))
                dst = jnp.broadcast_to(i_dst, (L,))
                for t in range(D // L):
                    row = slot * D + t * L + lanes
                    v = plsc.load_gather(ring, [row, col])
                    plsc.store_scatter(vals, [t * L + lanes, dst], v)

            first = idx_v[pl.ds(0, L)]
            for s in range(K):
                fire(first[s], s)

            def body(g, _):
                ch = idx_v[pl.ds(g * K, 2 * K)]
                for s in range(K):
                    drain_extract(ch[s], g * K + s, s)
                    fire(ch[K + s], s)
                return 0

            lax.fori_loop(0, b // K - 1, body, 0)
            last = idx_v[pl.ds(b - 2 * K, 2 * K)]
            for s in range(K):
                drain_extract(last[K + s], b - K + s, s)
            pltpu.sync_copy(vals, out_hbm.at[pl.ds(0, D), pl.ds(base, b)])

        one(xu, ut_r, out_u)
        one(xp, tt_r, out_p)
        one(xn, tt_r, out_n)

    u_t, p_t, n_t = gather3(x_user, x_track_pos, x_track_neg, ut, tt)
    return (u_t.T, p_t.T, n_t.T)
